# baseline (speedup 1.0000x reference)
import functools

import jax
import jax.numpy as jnp
from jax.experimental import pallas as pl
from jax.experimental.pallas import tpu as pltpu

_B = 128
_SPEC_LEN = 7
_SAMPLE_LEN = _SPEC_LEN + 1
_VOCAB = 100000
_INVALID = -1
_ROWS = _B * _SAMPLE_LEN
_V_BLK = 2048
_N_CHUNKS = (_VOCAB + _V_BLK - 1) // _V_BLK


def _rej_kernel(logits_ref, spec8_ref, out_ref, max_sc, idx_sc):
    c = pl.program_id(0)
    x = logits_ref[...]
    col = jax.lax.broadcasted_iota(jnp.int32, x.shape, 1) + c * _V_BLK
    x = jnp.where(col < _VOCAB, x, -jnp.inf)
    chunk_max = jnp.max(x, axis=1, keepdims=True)
    chunk_idx = jnp.min(
        jnp.where(x == chunk_max, col, _VOCAB), axis=1, keepdims=True
    )

    @pl.when(c == 0)
    def _():
        max_sc[...] = chunk_max
        idx_sc[...] = chunk_idx

    @pl.when(c > 0)
    def _():
        better = chunk_max > max_sc[...]
        max_sc[...] = jnp.where(better, chunk_max, max_sc[...])
        idx_sc[...] = jnp.where(better, chunk_idx, idx_sc[...])

    @pl.when(c == _N_CHUNKS - 1)
    def _():
        ids = idx_sc[...].reshape(_B, _SAMPLE_LEN)
        eq = (ids == spec8_ref[...]).astype(jnp.float32)
        ii = jax.lax.broadcasted_iota(jnp.int32, (_SAMPLE_LEN, _SAMPLE_LEN), 0)
        jj = jax.lax.broadcasted_iota(jnp.int32, (_SAMPLE_LEN, _SAMPLE_LEN), 1)
        tri = (ii < jj).astype(jnp.float32)
        cums = jax.lax.dot(eq, tri, precision=jax.lax.Precision.HIGHEST)
        jcol = jax.lax.broadcasted_iota(jnp.int32, (_B, _SAMPLE_LEN), 1)
        keep = cums.astype(jnp.int32) == jcol
        out_ref[...] = jnp.where(keep, ids, _INVALID)


@jax.jit
def kernel(logits, spec_token_ids):
    spec8 = jnp.concatenate(
        [spec_token_ids, jnp.full((_B, 1), _INVALID, jnp.int32)], axis=1
    )
    return pl.pallas_call(
        _rej_kernel,
        grid=(_N_CHUNKS,),
        in_specs=[
            pl.BlockSpec((_ROWS, _V_BLK), lambda c: (0, c)),
            pl.BlockSpec((_B, _SAMPLE_LEN), lambda c: (0, 0)),
        ],
        out_specs=pl.BlockSpec((_B, _SAMPLE_LEN), lambda c: (0, 0)),
        out_shape=jax.ShapeDtypeStruct((_B, _SAMPLE_LEN), jnp.int32),
        scratch_shapes=[
            pltpu.VMEM((_ROWS, 1), jnp.float32),
            pltpu.VMEM((_ROWS, 1), jnp.int32),
        ],
        compiler_params=pltpu.CompilerParams(
            dimension_semantics=("arbitrary",),
        ),
    )(logits, spec8)
